# Initial kernel scaffold; baseline (speedup 1.0000x reference)
import jax
import jax.numpy as jnp
from jax.experimental import pallas as pl


def kernel(pro, buf_grad, i):
    raise NotImplementedError("write your pallas kernel here")


# speedup vs baseline: 10.5093x; 10.5093x over previous
import jax
import jax.numpy as jnp
from jax.experimental import pallas as pl
from jax.experimental.pallas import tpu as pltpu

_BLK = 2048


def _bcast_body(idx_ref, vec_ref, out_ref):
    del idx_ref
    out_ref[...] = jnp.broadcast_to(vec_ref[0], out_ref.shape)


def kernel(pro, buf_grad, i):
    del pro
    batch = 16384
    num_emb = buf_grad.shape[1]
    idx = jnp.asarray(i, dtype=jnp.int32).reshape((1,))
    table3d = buf_grad.reshape(buf_grad.shape[0], 1, num_emb)
    grid_spec = pltpu.PrefetchScalarGridSpec(
        num_scalar_prefetch=1,
        grid=(batch // _BLK,),
        in_specs=[
            pl.BlockSpec((1, 1, num_emb), lambda g, idx_ref: (idx_ref[0], 0, 0)),
        ],
        out_specs=pl.BlockSpec((_BLK, num_emb), lambda g, idx_ref: (g, 0)),
    )
    return pl.pallas_call(
        _bcast_body,
        grid_spec=grid_spec,
        out_shape=jax.ShapeDtypeStruct((batch, num_emb), jnp.float32),
    )(idx, table3d)


# speedup vs baseline: 13.0132x; 1.2383x over previous
import jax
import jax.numpy as jnp
from jax.experimental import pallas as pl
from jax.experimental.pallas import tpu as pltpu

_BLK = 8192


def _bcast_body(idx_ref, vec_ref, out_ref):
    del idx_ref
    out_ref[...] = jnp.broadcast_to(vec_ref[0], out_ref.shape)


def kernel(pro, buf_grad, i):
    del pro
    batch = 16384
    num_emb = buf_grad.shape[1]
    idx = jnp.asarray(i, dtype=jnp.int32).reshape((1,))
    table3d = buf_grad.reshape(buf_grad.shape[0], 1, num_emb)
    grid_spec = pltpu.PrefetchScalarGridSpec(
        num_scalar_prefetch=1,
        grid=(batch // _BLK,),
        in_specs=[
            pl.BlockSpec((1, 1, num_emb), lambda g, idx_ref: (idx_ref[0], 0, 0)),
        ],
        out_specs=pl.BlockSpec((_BLK, num_emb), lambda g, idx_ref: (g, 0)),
    )
    return pl.pallas_call(
        _bcast_body,
        grid_spec=grid_spec,
        out_shape=jax.ShapeDtypeStruct((batch, num_emb), jnp.float32),
    )(idx, table3d)
